# Initial kernel scaffold; baseline (speedup 1.0000x reference)
import functools

import jax
import jax.numpy as jnp
from jax import lax
from jax.experimental import pallas as pl
from jax.experimental.pallas import tpu as pltpu
from jax.experimental.pallas import tpu_sc as plsc

N = 10000
D = 128
E = 320000
NCH = 157
CPR = 2501


def _tc_prep(x, W0, W1, A0, A1):
    blk = 1000
    grid = (N // blk,)

    def body(x_ref, w0_ref, w1_ref, a0_ref, a1_ref, feat_ref, elr_ref):
        xb = x_ref[...]
        f0 = jnp.dot(xb, w0_ref[...], preferred_element_type=jnp.float32)
        f1 = jnp.dot(xb, w1_ref[...], preferred_element_type=jnp.float32)
        feat_ref[0] = f0
        feat_ref[1] = f1
        elr_ref[0] = jnp.dot(f0, a0_ref[...], preferred_element_type=jnp.float32)
        elr_ref[1] = jnp.dot(f1, a1_ref[...], preferred_element_type=jnp.float32)

    return pl.pallas_call(
        body,
        grid=grid,
        in_specs=[
            pl.BlockSpec((blk, D), lambda i: (i, 0)),
            pl.BlockSpec((D, D), lambda i: (0, 0)),
            pl.BlockSpec((D, D), lambda i: (0, 0)),
            pl.BlockSpec((D, 2), lambda i: (0, 0)),
            pl.BlockSpec((D, 2), lambda i: (0, 0)),
        ],
        out_specs=[
            pl.BlockSpec((2, blk, D), lambda i: (0, i, 0)),
            pl.BlockSpec((2, blk, 2), lambda i: (0, i, 0)),
        ],
        out_shape=[
            jax.ShapeDtypeStruct((2, N, D), jnp.float32),
            jax.ShapeDtypeStruct((2, N, 2), jnp.float32),
        ],
    )(x, W0, W1, A0, A1)


def _tc_combine(rst, b0, b1):
    blk = 1000
    grid = (N // blk,)

    def body(ra_ref, rb_ref, b0_ref, b1_ref, o_ref):
        o_ref[...] = ra_ref[...] + rb_ref[...] + b0_ref[...] + b1_ref[...]

    return pl.pallas_call(
        body,
        grid=grid,
        in_specs=[
            pl.BlockSpec((blk, D), lambda i: (i, 0)),
            pl.BlockSpec((blk, D), lambda i: (i + N // blk, 0)),
            pl.BlockSpec((1, D), lambda i: (0, 0)),
            pl.BlockSpec((1, D), lambda i: (0, 0)),
        ],
        out_specs=pl.BlockSpec((blk, D), lambda i: (i, 0)),
        out_shape=jax.ShapeDtypeStruct((N, D), jnp.float32),
    )(rst, rst, b0, b1)


def _sc_body(feat_h, elr_h, src_h, dst_h, x_h, out_h,
             elr_v, src_v, dst_v, ee_v, dn_v, al_v, rows_v, z_v,
             denom_s, acc_s):
    cid = lax.axis_index("c")
    sid = lax.axis_index("s")
    nch = jnp.where(sid < 4, 157, 156)
    ch0 = 156 * sid + jnp.minimum(sid, 4)
    base = jnp.minimum(640 * sid, 9360)

    pltpu.sync_copy(elr_h.at[pl.ds(cid * N, N)], elr_v)
    pltpu.sync_copy(src_h.at[pl.ds(cid * CPR + ch0, NCH)], src_v)
    pltpu.sync_copy(dst_h.at[pl.ds(cid * CPR + ch0, NCH)], dst_v)

    def zset(i, c):
        z_v[pl.ds(16 * i, 16)] = jnp.zeros((16,), jnp.float32)
        return c
    lax.fori_loop(0, 40, zset, 0)
    pltpu.sync_copy(z_v, denom_s.at[pl.ds(base, 640)])
    pltpu.sync_copy(x_h.at[pl.ds(base, 640)], acc_s.at[pl.ds(base, 640)])
    plsc.subcore_barrier()

    czero = jnp.zeros((16,), jnp.int32)
    cone = jnp.ones((16,), jnp.int32)
    off = jnp.full((16,), cid * N, jnp.int32)

    def pass_a(j, c):
        for i in range(8):
            sl = pl.ds(16 * i, 16)
            sv = src_v[j, sl]
            dv = dst_v[j, sl]
            el = plsc.load_gather(elr_v, [sv, czero])
            er = plsc.load_gather(elr_v, [dv, cone])
            e = el + er
            e = jnp.where(e > 0, e, 0.2 * e)
            ee_v[j, sl] = jnp.exp(e)
            src_v[j, sl] = sv + off
        pltpu.sync_copy(ee_v.at[j], denom_s.at[dst_v.at[j]], add=True)
        return c
    lax.fori_loop(0, nch, pass_a, 0)
    plsc.subcore_barrier()
    pltpu.sync_copy(denom_s, dn_v)

    def pass_b(j, c):
        for i in range(8):
            sl = pl.ds(16 * i, 16)
            dv = dst_v[j, sl]
            dn = plsc.load_gather(dn_v, [dv])
            al_v[sl] = ee_v[j, sl] / jnp.maximum(dn, 1e-9)
        pltpu.sync_copy(feat_h.at[src_v.at[j]], rows_v)

        def scale(k, cc):
            a = plsc.load_gather(al_v, [jnp.full((16,), k, jnp.int32)])
            for t in range(8):
                slt = pl.ds(16 * t, 16)
                rows_v[k, slt] = rows_v[k, slt] * a
            return cc
        lax.fori_loop(0, 128, scale, 0)
        pltpu.sync_copy(rows_v, acc_s.at[dst_v.at[j]], add=True)
        return c
    lax.fori_loop(0, nch, pass_b, 0)
    plsc.subcore_barrier()

    pltpu.sync_copy(acc_s.at[pl.ds(base, 640)],
                    out_h.at[pl.ds(cid * N + base, 640)])


@functools.cache
def _sc_kernel():
    mesh = plsc.VectorSubcoreMesh(core_axis_name="c", subcore_axis_name="s",
                                  num_cores=2, num_subcores=16)
    return pl.kernel(
        _sc_body,
        out_type=jax.ShapeDtypeStruct((2 * N, D), jnp.float32),
        mesh=mesh,
        scratch_types=[
            pltpu.VMEM((N, 2), jnp.float32),
            pltpu.VMEM((NCH, 128), jnp.int32),
            pltpu.VMEM((NCH, 128), jnp.int32),
            pltpu.VMEM((NCH, 128), jnp.float32),
            pltpu.VMEM((N,), jnp.float32),
            pltpu.VMEM((128,), jnp.float32),
            pltpu.VMEM((128, 128), jnp.float32),
            pltpu.VMEM((640,), jnp.float32),
            pltpu.VMEM_SHARED((N,), jnp.float32),
            pltpu.VMEM_SHARED((N, D), jnp.float32),
        ],
    )


def _pad_idx(row):
    return jnp.concatenate(
        [row.reshape(2500, 128), jnp.zeros((1, 128), jnp.int32)], axis=0)


@jax.jit
def kernel(x, edge_index_rel0, edge_index_rel1,
           W0, attn_l0, attn_r0, bias0,
           W1, attn_l1, attn_r1, bias1):
    A0 = jnp.stack([attn_l0, attn_r0], axis=1)
    A1 = jnp.stack([attn_l1, attn_r1], axis=1)
    feat, elr = _tc_prep(x, W0, W1, A0, A1)
    srcp = jnp.concatenate(
        [_pad_idx(edge_index_rel0[0]), _pad_idx(edge_index_rel1[0])], axis=0)
    dstp = jnp.concatenate(
        [_pad_idx(edge_index_rel0[1]), _pad_idx(edge_index_rel1[1])], axis=0)
    rst = _sc_kernel()(feat.reshape(2 * N, D), elr.reshape(2 * N, 2),
                       srcp, dstp, x)
    return _tc_combine(rst, bias0.reshape(1, D), bias1.reshape(1, D))


# speedup vs baseline: 23.7141x; 23.7141x over previous
import functools

import jax
import jax.numpy as jnp
from jax import lax
from jax.experimental import pallas as pl
from jax.experimental.pallas import tpu as pltpu
from jax.experimental.pallas import tpu_sc as plsc

N = 10000
D = 128
E = 320000


def _tc_prep(x, W0, W1, A0, A1):
    blk = 1000
    grid = (N // blk,)

    def body(x_ref, w0_ref, w1_ref, a0_ref, a1_ref, feat_ref, elr_ref):
        xb = x_ref[...]
        f0 = jnp.dot(xb, w0_ref[...], preferred_element_type=jnp.float32)
        f1 = jnp.dot(xb, w1_ref[...], preferred_element_type=jnp.float32)
        feat_ref[0] = f0
        feat_ref[1] = f1
        elr_ref[0] = jnp.dot(f0, a0_ref[...], preferred_element_type=jnp.float32)
        elr_ref[1] = jnp.dot(f1, a1_ref[...], preferred_element_type=jnp.float32)

    return pl.pallas_call(
        body,
        grid=grid,
        in_specs=[
            pl.BlockSpec((blk, D), lambda i: (i, 0)),
            pl.BlockSpec((D, D), lambda i: (0, 0)),
            pl.BlockSpec((D, D), lambda i: (0, 0)),
            pl.BlockSpec((D, 2), lambda i: (0, 0)),
            pl.BlockSpec((D, 2), lambda i: (0, 0)),
        ],
        out_specs=[
            pl.BlockSpec((2, blk, D), lambda i: (0, i, 0)),
            pl.BlockSpec((2, blk, 2), lambda i: (0, i, 0)),
        ],
        out_shape=[
            jax.ShapeDtypeStruct((2, N, D), jnp.float32),
            jax.ShapeDtypeStruct((2, N, 2), jnp.float32),
        ],
    )(x, W0, W1, A0, A1)


def _tc_combine(rst, b0, b1):
    blk = 1000
    grid = (N // blk,)

    def body(ra_ref, rb_ref, b0_ref, b1_ref, o_ref):
        o_ref[...] = ra_ref[...] + rb_ref[...] + b0_ref[...] + b1_ref[...]

    return pl.pallas_call(
        body,
        grid=grid,
        in_specs=[
            pl.BlockSpec((blk, D), lambda i: (i, 0)),
            pl.BlockSpec((blk, D), lambda i: (i + N // blk, 0)),
            pl.BlockSpec((1, D), lambda i: (0, 0)),
            pl.BlockSpec((1, D), lambda i: (0, 0)),
        ],
        out_specs=pl.BlockSpec((blk, D), lambda i: (i, 0)),
        out_shape=jax.ShapeDtypeStruct((N, D), jnp.float32),
    )(rst, rst, b0, b1)


def _sc_body(feat_h, elr_h, src_h, dst_h, x_h, out_h,
             elr_v, src_g, dst_g, ee_row, rows_v, z_v,
             denom_s, acc_s):
    cid = lax.axis_index("c")
    sid = lax.axis_index("s")
    nch = jnp.where(sid < 4, 157, 156)
    plane = cid * 2560 + sid * 160
    base = jnp.minimum(640 * sid, 9360)

    pltpu.sync_copy(elr_h.at[pl.ds(cid * 20480, 20480)], elr_v)

    def zset(i, c):
        z_v[pl.ds(16 * i, 16)] = jnp.zeros((16,), jnp.float32)
        return c
    lax.fori_loop(0, 40, zset, 0)
    pltpu.sync_copy(z_v, denom_s.at[pl.ds(base, 640)])

    def zrow(i, c):
        rows_v[i // 8, pl.ds(16 * (i % 8), 16)] = jnp.zeros((16,), jnp.float32)
        return c
    lax.fori_loop(0, 512, zrow, 0)
    def zacc(b, c):
        pltpu.sync_copy(rows_v.at[pl.ds(0, 64)],
                        acc_s.at[pl.ds(base + 64 * b, 64)])
        return c
    lax.fori_loop(0, 10, zacc, 0)
    plsc.subcore_barrier()

    off = jnp.full((16,), cid * N, jnp.int32)

    def chunk(jj, c):
        for i in range(8):
            sl = pl.ds(16 * i, 16)
            sv = src_g[jj, sl]
            dv = dst_g[jj, sl]
            el = plsc.load_gather(elr_v, [sv + sv])
            er = plsc.load_gather(elr_v, [dv + dv + 1])
            e = el + er
            e = jnp.where(e > 0, e, 0.2 * e)
            ee_row[sl] = jnp.exp(e)
            src_g[jj, sl] = sv + off
        pltpu.sync_copy(ee_row, denom_s.at[dst_g.at[jj]], add=True)
        pltpu.sync_copy(feat_h.at[src_g.at[jj]], rows_v)

        def scale(k, cc):
            a = plsc.load_gather(ee_row, [jnp.full((16,), k, jnp.int32)])
            for t in range(8):
                slt = pl.ds(16 * t, 16)
                rows_v[k, slt] = rows_v[k, slt] * a
            return cc
        lax.fori_loop(0, 128, scale, 0)
        pltpu.sync_copy(rows_v, acc_s.at[dst_g.at[jj]], add=True)
        return c

    def group(g, njj, c):
        pltpu.sync_copy(src_h.at[pl.ds(plane + 16 * g, 16)], src_g)
        pltpu.sync_copy(dst_h.at[pl.ds(plane + 16 * g, 16)], dst_g)
        return lax.fori_loop(0, njj, chunk, c)

    lax.fori_loop(0, 9, lambda g, c: group(g, 16, c), 0)
    group(9, nch - 144, 0)
    plsc.subcore_barrier()

    pltpu.sync_copy(denom_s.at[pl.ds(base, 640)], z_v)

    def recip(i, c):
        sl = pl.ds(16 * i, 16)
        z_v[sl] = 1.0 / jnp.maximum(z_v[sl], 1e-9)
        return c
    lax.fori_loop(0, 40, recip, 0)

    def wb_block(b, c):
        pltpu.sync_copy(acc_s.at[pl.ds(base + 64 * b, 64)],
                        rows_v.at[pl.ds(0, 64)])
        pltpu.sync_copy(x_h.at[pl.ds(base + 64 * b, 64)],
                        rows_v.at[pl.ds(64, 64)])

        def norm(r, cc):
            a = plsc.load_gather(z_v, [jnp.full((16,), 64 * b + r, jnp.int32)])
            for t in range(8):
                slt = pl.ds(16 * t, 16)
                rows_v[r, slt] = rows_v[r, slt] * a + rows_v[64 + r, slt]
            return cc
        lax.fori_loop(0, 64, norm, 0)
        pltpu.sync_copy(rows_v.at[pl.ds(0, 64)],
                        out_h.at[pl.ds(cid * N + base + 64 * b, 64)])
        return c
    lax.fori_loop(0, 10, wb_block, 0)


@functools.cache
def _sc_kernel():
    mesh = plsc.VectorSubcoreMesh(core_axis_name="c", subcore_axis_name="s",
                                  num_cores=2, num_subcores=16)
    return pl.kernel(
        _sc_body,
        out_type=jax.ShapeDtypeStruct((2 * N, D), jnp.float32),
        mesh=mesh,
        compiler_params=pltpu.CompilerParams(needs_layout_passes=False),
        scratch_types=[
            pltpu.VMEM((20480,), jnp.float32),
            pltpu.VMEM((16, 128), jnp.int32),
            pltpu.VMEM((16, 128), jnp.int32),
            pltpu.VMEM((128,), jnp.float32),
            pltpu.VMEM((128, 128), jnp.float32),
            pltpu.VMEM((640,), jnp.float32),
            pltpu.VMEM_SHARED((N,), jnp.float32),
            pltpu.VMEM_SHARED((N, D), jnp.float32),
        ],
    )


def _pack_idx(row):
    chunks = row.reshape(2500, 128)
    zpad = jnp.zeros((160, 128), jnp.int32)
    parts = []
    for t in range(16):
        n = 157 if t < 4 else 156
        c0 = 156 * t + min(t, 4)
        parts.append(chunks[c0:c0 + n])
        parts.append(zpad[:160 - n])
    return jnp.concatenate(parts, axis=0)


@jax.jit
def kernel(x, edge_index_rel0, edge_index_rel1,
           W0, attn_l0, attn_r0, bias0,
           W1, attn_l1, attn_r1, bias1):
    A0 = jnp.stack([attn_l0, attn_r0], axis=1)
    A1 = jnp.stack([attn_l1, attn_r1], axis=1)
    feat, elr = _tc_prep(x, W0, W1, A0, A1)
    elrp = jnp.pad(elr.reshape(2, 2 * N), ((0, 0), (0, 480))).reshape(40960)
    srcp = jnp.concatenate(
        [_pack_idx(edge_index_rel0[0]), _pack_idx(edge_index_rel1[0])], axis=0)
    dstp = jnp.concatenate(
        [_pack_idx(edge_index_rel0[1]), _pack_idx(edge_index_rel1[1])], axis=0)
    rst = _sc_kernel()(feat.reshape(2 * N, D), elrp, srcp, dstp, x)
    return _tc_combine(rst, bias0.reshape(1, D), bias1.reshape(1, D))
